# Initial kernel scaffold; baseline (speedup 1.0000x reference)
import jax
import jax.numpy as jnp
from jax.experimental import pallas as pl


def kernel(x, edge_index, W1, b1, W2, b2, Wout, bout):
    raise NotImplementedError("write your pallas kernel here")


# speedup vs baseline: 9.7912x; 9.7912x over previous
import functools

import jax
import jax.numpy as jnp
from jax import lax
from jax.experimental import pallas as pl
from jax.experimental.pallas import tpu as pltpu
from jax.experimental.pallas import tpu_sc as plsc

N = 10000
D = 128
E = 320000
H = 128
OUT = 128

NW = 32
B = 128
BPT = 80
ROWS = NW * BPT
EP = ROWS * B

NSUB = 16
RPS = 640
NP = NSUB * RPS
WCH = RPS // B
NPB = NP // 2


def _zero_rows(ref, nrows):
    zeros16 = jnp.zeros((16,), jnp.float32)

    def body(r, _):
        for o in range(8):
            ref[r, pl.ds(o * 16, 16)] = zeros16
        return 0

    lax.fori_loop(0, nrows, body, 0)


def _deg_body(sidx_hbm, didx_hbm, out_hbm, sidx_v, didx_v, acc_v):
    cid = lax.axis_index("c")
    sid = lax.axis_index("s")
    wid = sid * 2 + cid

    def zb(i, _):
        acc_v[pl.ds(i * 16, 16)] = jnp.zeros((16,), jnp.float32)
        return 0

    lax.fori_loop(0, (2 * NP) // 16, zb, 0)

    pltpu.sync_copy(sidx_hbm.at[pl.ds(wid * BPT, BPT)], sidx_v)
    pltpu.sync_copy(didx_hbm.at[pl.ds(wid * BPT, BPT)], didx_v)

    ones16 = jnp.ones((16,), jnp.float32)

    def body(j, _):
        for o in range(8):
            plsc.addupdate_scatter(acc_v, [sidx_v[j, pl.ds(o * 16, 16)]], ones16)
            plsc.addupdate_scatter(acc_v, [didx_v[j, pl.ds(o * 16, 16)]], ones16)
        return 0

    lax.fori_loop(0, BPT, body, 0)
    pltpu.sync_copy(acc_v, out_hbm.at[pl.ds(wid * 2 * NP, 2 * NP)])


def _ep_body(q_hbm, sidx_hbm, didx_hbm, out_hbm, sidx_v, didx_v, rows_v, acc_sh, sem):
    cid = lax.axis_index("c")
    sid = lax.axis_index("s")
    wid = sid * 2 + cid

    _zero_rows(rows_v, B)
    for k in range(WCH):
        pltpu.sync_copy(rows_v, acc_sh.at[pl.ds(sid * RPS + k * B, B)])
    plsc.subcore_barrier()

    pltpu.sync_copy(sidx_hbm.at[pl.ds(wid * BPT, BPT)], sidx_v)
    pltpu.sync_copy(didx_hbm.at[pl.ds(wid * BPT, BPT)], didx_v)

    def body(j, _):
        pltpu.async_copy(q_hbm.at[sidx_v.at[j]], rows_v, sem).wait()
        pltpu.sync_copy(rows_v, acc_sh.at[didx_v.at[j]], add=True)
        return 0

    lax.fori_loop(0, BPT, body, 0)
    plsc.subcore_barrier()

    for k in range(WCH):
        sl = pl.ds(sid * RPS + k * B, B)
        pltpu.sync_copy(acc_sh.at[sl], rows_v)
        pltpu.sync_copy(rows_v, out_hbm.at[cid, sl])


def _make_sc_kernels():
    mesh = plsc.VectorSubcoreMesh(core_axis_name="c", subcore_axis_name="s")
    params = pltpu.CompilerParams(needs_layout_passes=False)
    deg = pl.kernel(
        _deg_body,
        out_type=jax.ShapeDtypeStruct((NW * 2 * NP,), jnp.float32),
        mesh=mesh,
        compiler_params=params,
        scratch_types=[
            pltpu.VMEM((BPT, B), jnp.int32),
            pltpu.VMEM((BPT, B), jnp.int32),
            pltpu.VMEM((2 * NP,), jnp.float32),
        ],
    )
    ep = pl.kernel(
        _ep_body,
        out_type=jax.ShapeDtypeStruct((2, NP, 128), jnp.float32),
        mesh=mesh,
        compiler_params=params,
        scratch_types=[
            pltpu.VMEM((BPT, B), jnp.int32),
            pltpu.VMEM((BPT, B), jnp.int32),
            pltpu.VMEM((B, 128), jnp.float32),
            pltpu.VMEM_SHARED((NP, 128), jnp.float32),
            pltpu.SemaphoreType.DMA,
        ],
    )
    return deg, ep


_PREC = jax.lax.Precision.HIGHEST


def _tc1_body(degp_ref, x_ref, w1_ref, q1_ref, ns_ref, nd_ref):
    deg = jnp.sum(degp_ref[...], axis=0)
    ns = lax.rsqrt(jnp.maximum(deg[0], 1.0))[:, None]
    nd = lax.rsqrt(jnp.maximum(deg[1], 1.0))[:, None]
    ns_ref[...] = ns
    nd_ref[...] = nd
    q1_ref[...] = jnp.dot(x_ref[...] * ns, w1_ref[...],
                          preferred_element_type=jnp.float32, precision=_PREC)


def _tc2_body(agg_ref, nd_ref, ns_ref, b_ref, w_ref, h_ref, q_ref):
    s = agg_ref[0] + agg_ref[1]
    h = jnp.maximum(s * nd_ref[...] + b_ref[...], 0.0)
    h_ref[...] = h
    q_ref[...] = jnp.dot(h * ns_ref[...], w_ref[...],
                         preferred_element_type=jnp.float32, precision=_PREC)


def _tc3_body(agg_ref, nd_ref, b_ref, h1_ref, wout_ref, z_ref):
    s = agg_ref[0] + agg_ref[1]
    h2 = jnp.maximum(s * nd_ref[...] + b_ref[...], 0.0)
    z_ref[...] = (
        jnp.dot(h1_ref[...], wout_ref[:H], preferred_element_type=jnp.float32,
                precision=_PREC)
        + jnp.dot(h2, wout_ref[H:], preferred_element_type=jnp.float32,
                  precision=_PREC)
    )


def _tc4_body(agg_ref, b_ref, out_ref):
    out_ref[...] = agg_ref[0] + agg_ref[1] + b_ref[...]


def _row_blk(i):
    return (i, 0)


def _agg_blk(i):
    return (0, i, 0)


def _full_blk(i):
    return (0, 0)


_ROWS_SPEC = pl.BlockSpec((NPB, 128), _row_blk)
_COL_SPEC = pl.BlockSpec((NPB, 1), _row_blk)
_AGG_SPEC = pl.BlockSpec((2, NPB, 128), _agg_blk)
_B_SPEC = pl.BlockSpec((1, 128), _full_blk)


def _tc1(degp, x_p, W1):
    return pl.pallas_call(
        _tc1_body,
        grid=(2,),
        in_specs=[
            pl.BlockSpec((NW, 2, NPB), lambda i: (0, 0, i)),
            _ROWS_SPEC,
            pl.BlockSpec((128, 128), _full_blk),
        ],
        out_specs=[_ROWS_SPEC, _COL_SPEC, _COL_SPEC],
        out_shape=[
            jax.ShapeDtypeStruct((NP, 128), jnp.float32),
            jax.ShapeDtypeStruct((NP, 1), jnp.float32),
            jax.ShapeDtypeStruct((NP, 1), jnp.float32),
        ],
    )(degp, x_p, W1)


def _tc2(agg, nd, ns, b, W):
    return pl.pallas_call(
        _tc2_body,
        grid=(2,),
        in_specs=[_AGG_SPEC, _COL_SPEC, _COL_SPEC, _B_SPEC,
                  pl.BlockSpec((128, 128), _full_blk)],
        out_specs=[_ROWS_SPEC, _ROWS_SPEC],
        out_shape=[
            jax.ShapeDtypeStruct((NP, 128), jnp.float32),
            jax.ShapeDtypeStruct((NP, 128), jnp.float32),
        ],
    )(agg, nd, ns, b, W)


def _tc3(agg, nd, b, h1, Wout):
    return pl.pallas_call(
        _tc3_body,
        grid=(2,),
        in_specs=[_AGG_SPEC, _COL_SPEC, _B_SPEC, _ROWS_SPEC,
                  pl.BlockSpec((2 * H, 128), _full_blk)],
        out_specs=_ROWS_SPEC,
        out_shape=jax.ShapeDtypeStruct((NP, 128), jnp.float32),
    )(agg, nd, b, h1, Wout)


def _tc4(agg, b):
    return pl.pallas_call(
        _tc4_body,
        grid=(2,),
        in_specs=[_AGG_SPEC, _B_SPEC],
        out_specs=_ROWS_SPEC,
        out_shape=jax.ShapeDtypeStruct((NP, 128), jnp.float32),
    )(agg, b)


def kernel(x, edge_index, W1, b1, W2, b2, Wout, bout):
    src = edge_index[0]
    dst = edge_index[1]

    pad = (N + (jnp.arange(EP - E, dtype=jnp.int32) % (NP - N))).astype(jnp.int32)
    src_p = jnp.concatenate([src, pad]).reshape(ROWS, B)
    dst_p = jnp.concatenate([dst, pad]).reshape(ROWS, B)
    didxN = dst_p + NP
    x_p = jnp.pad(x, ((0, NP - N), (0, 0)))

    deg_k, ep_k = _make_sc_kernels()

    degp = deg_k(src_p, didxN).reshape(NW, 2, NP)
    q1, ns, nd = _tc1(degp, x_p, W1)
    agg1 = ep_k(q1, src_p, dst_p)
    h1, q2 = _tc2(agg1, nd, ns, b1.reshape(1, H), W2)
    agg2 = ep_k(q2, src_p, dst_p)
    z = _tc3(agg2, nd, b2.reshape(1, H), h1, Wout)
    agg3 = ep_k(z, src_p, dst_p)
    outp = _tc4(agg3, bout.reshape(1, OUT))
    return outp[:N]


# speedup vs baseline: 14.5678x; 1.4879x over previous
import functools

import jax
import jax.numpy as jnp
from jax import lax
from jax.experimental import pallas as pl
from jax.experimental.pallas import tpu as pltpu
from jax.experimental.pallas import tpu_sc as plsc

N = 10000
D = 128
E = 320000
H = 128
OUT = 128

NW = 32
B = 128
BPT = 80
ROWS = NW * BPT
EP = ROWS * B

NSUB = 16
RPS = 640
NP = NSUB * RPS
WCH = RPS // B
NPB = NP // 2


def _zero_rows(ref, nrows):
    zeros16 = jnp.zeros((16,), jnp.float32)

    def body(r, _):
        for o in range(8):
            ref[r, pl.ds(o * 16, 16)] = zeros16
        return 0

    lax.fori_loop(0, nrows, body, 0)


def _deg_body(sidx_hbm, didx_hbm, out_hbm, sidx_v, didx_v, acc_v):
    cid = lax.axis_index("c")
    sid = lax.axis_index("s")
    wid = sid * 2 + cid

    def zb(i, _):
        acc_v[pl.ds(i * 16, 16)] = jnp.zeros((16,), jnp.float32)
        return 0

    lax.fori_loop(0, (2 * NP) // 16, zb, 0)

    pltpu.sync_copy(sidx_hbm.at[pl.ds(wid * BPT, BPT)], sidx_v)
    pltpu.sync_copy(didx_hbm.at[pl.ds(wid * BPT, BPT)], didx_v)

    ones16 = jnp.ones((16,), jnp.float32)

    def body(j, _):
        for o in range(8):
            plsc.addupdate_scatter(acc_v, [sidx_v[j, pl.ds(o * 16, 16)]], ones16)
            plsc.addupdate_scatter(acc_v, [didx_v[j, pl.ds(o * 16, 16)]], ones16)
        return 0

    lax.fori_loop(0, BPT, body, 0)
    pltpu.sync_copy(acc_v, out_hbm.at[pl.ds(wid * 2 * NP, 2 * NP)])


IG = 16
NCH = BPT // IG


def _ep_body(q_hbm, sidx_hbm, didx_hbm, out_hbm, sidx_v, di0, di1,
             rv0, rv1, acc_sh, gs0, gs1, isem):
    cid = lax.axis_index("c")
    sid = lax.axis_index("s")
    wid = sid * 2 + cid
    rows = (rv0, rv1)
    gsems = (gs0, gs1)
    dibufs = (di0, di1)

    _zero_rows(rv0, B)
    for k in range(WCH):
        pltpu.sync_copy(rv0, acc_sh.at[pl.ds(sid * RPS + k * B, B)])
    plsc.subcore_barrier()

    pltpu.sync_copy(sidx_hbm.at[pl.ds(wid * BPT, BPT)], sidx_v)
    pltpu.sync_copy(didx_hbm.at[pl.ds(wid * BPT, IG)], di0)

    def gather(t, b):
        pltpu.async_copy(q_hbm.at[sidx_v.at[t]], rows[b], gsems[b])

    def gwait(b):
        pltpu.make_async_copy(q_hbm.at[sidx_v.at[0]], rows[b], gsems[b]).wait()

    def scatter(r, di, b):
        pltpu.sync_copy(rows[b], acc_sh.at[di.at[r]], add=True)

    def load_chunk(c):
        pltpu.async_copy(didx_hbm.at[pl.ds(wid * BPT + c * IG, IG)],
                         dibufs[c % 2], isem)

    def chunk_wait(c):
        pltpu.make_async_copy(didx_hbm.at[pl.ds(0, IG)], dibufs[c % 2],
                              isem).wait()

    gather(0, 0)
    gather(1, 1)
    load_chunk(1)

    for c in range(NCH):
        di = dibufs[c % 2]
        base = c * IG
        if c >= 1:
            chunk_wait(c)
            if c + 1 < NCH:
                load_chunk(c + 1)

        last = c == NCH - 1
        hi = IG // 2 - 1 if last else IG // 2

        def cpair(i, _, di=di, base=base):
            r = 2 * i
            t = base + r
            gwait(0); scatter(r, di, 0); gather(t + 2, 0)
            gwait(1); scatter(r + 1, di, 1); gather(t + 3, 1)
            return 0

        lax.fori_loop(0, hi, cpair, 0)
        if last:
            gwait(0); scatter(IG - 2, di, 0)
            gwait(1); scatter(IG - 1, di, 1)

    plsc.subcore_barrier()

    for k in range(WCH):
        sl = pl.ds(sid * RPS + k * B, B)
        pltpu.sync_copy(acc_sh.at[sl], rv0)
        pltpu.sync_copy(rv0, out_hbm.at[cid, sl])


def _make_sc_kernels():
    mesh = plsc.VectorSubcoreMesh(core_axis_name="c", subcore_axis_name="s")
    params = pltpu.CompilerParams(needs_layout_passes=False)
    deg = pl.kernel(
        _deg_body,
        out_type=jax.ShapeDtypeStruct((NW * 2 * NP,), jnp.float32),
        mesh=mesh,
        compiler_params=params,
        scratch_types=[
            pltpu.VMEM((BPT, B), jnp.int32),
            pltpu.VMEM((BPT, B), jnp.int32),
            pltpu.VMEM((2 * NP,), jnp.float32),
        ],
    )
    ep = pl.kernel(
        _ep_body,
        out_type=jax.ShapeDtypeStruct((2, NP, 128), jnp.float32),
        mesh=mesh,
        compiler_params=params,
        scratch_types=(
            [pltpu.VMEM((BPT, B), jnp.int32)]
            + [pltpu.VMEM((IG, B), jnp.int32)] * 2
            + [pltpu.VMEM((B, 128), jnp.float32)] * 2
            + [pltpu.VMEM_SHARED((NP, 128), jnp.float32)]
            + [pltpu.SemaphoreType.DMA] * 3
        ),
    )
    return deg, ep


_PREC = jax.lax.Precision.HIGHEST


def _tc1_body(degp_ref, x_ref, w1_ref, q1_ref, ns_ref, nd_ref):
    deg = jnp.sum(degp_ref[...], axis=0)
    ns = lax.rsqrt(jnp.maximum(deg[0], 1.0))[:, None]
    nd = lax.rsqrt(jnp.maximum(deg[1], 1.0))[:, None]
    ns_ref[...] = ns
    nd_ref[...] = nd
    q1_ref[...] = jnp.dot(x_ref[...] * ns, w1_ref[...],
                          preferred_element_type=jnp.float32, precision=_PREC)


def _tc2_body(agg_ref, nd_ref, ns_ref, b_ref, w_ref, h_ref, q_ref):
    s = agg_ref[0] + agg_ref[1]
    h = jnp.maximum(s * nd_ref[...] + b_ref[...], 0.0)
    h_ref[...] = h
    q_ref[...] = jnp.dot(h * ns_ref[...], w_ref[...],
                         preferred_element_type=jnp.float32, precision=_PREC)


def _tc3_body(agg_ref, nd_ref, b_ref, h1_ref, wout_ref, z_ref):
    s = agg_ref[0] + agg_ref[1]
    h2 = jnp.maximum(s * nd_ref[...] + b_ref[...], 0.0)
    z_ref[...] = (
        jnp.dot(h1_ref[...], wout_ref[:H], preferred_element_type=jnp.float32,
                precision=_PREC)
        + jnp.dot(h2, wout_ref[H:], preferred_element_type=jnp.float32,
                  precision=_PREC)
    )


def _tc4_body(agg_ref, b_ref, out_ref):
    out_ref[...] = agg_ref[0] + agg_ref[1] + b_ref[...]


def _row_blk(i):
    return (i, 0)


def _agg_blk(i):
    return (0, i, 0)


def _full_blk(i):
    return (0, 0)


_ROWS_SPEC = pl.BlockSpec((NPB, 128), _row_blk)
_COL_SPEC = pl.BlockSpec((NPB, 1), _row_blk)
_AGG_SPEC = pl.BlockSpec((2, NPB, 128), _agg_blk)
_B_SPEC = pl.BlockSpec((1, 128), _full_blk)


def _tc1(degp, x_p, W1):
    return pl.pallas_call(
        _tc1_body,
        grid=(2,),
        in_specs=[
            pl.BlockSpec((NW, 2, NPB), lambda i: (0, 0, i)),
            _ROWS_SPEC,
            pl.BlockSpec((128, 128), _full_blk),
        ],
        out_specs=[_ROWS_SPEC, _COL_SPEC, _COL_SPEC],
        out_shape=[
            jax.ShapeDtypeStruct((NP, 128), jnp.float32),
            jax.ShapeDtypeStruct((NP, 1), jnp.float32),
            jax.ShapeDtypeStruct((NP, 1), jnp.float32),
        ],
    )(degp, x_p, W1)


def _tc2(agg, nd, ns, b, W):
    return pl.pallas_call(
        _tc2_body,
        grid=(2,),
        in_specs=[_AGG_SPEC, _COL_SPEC, _COL_SPEC, _B_SPEC,
                  pl.BlockSpec((128, 128), _full_blk)],
        out_specs=[_ROWS_SPEC, _ROWS_SPEC],
        out_shape=[
            jax.ShapeDtypeStruct((NP, 128), jnp.float32),
            jax.ShapeDtypeStruct((NP, 128), jnp.float32),
        ],
    )(agg, nd, ns, b, W)


def _tc3(agg, nd, b, h1, Wout):
    return pl.pallas_call(
        _tc3_body,
        grid=(2,),
        in_specs=[_AGG_SPEC, _COL_SPEC, _B_SPEC, _ROWS_SPEC,
                  pl.BlockSpec((2 * H, 128), _full_blk)],
        out_specs=_ROWS_SPEC,
        out_shape=jax.ShapeDtypeStruct((NP, 128), jnp.float32),
    )(agg, nd, b, h1, Wout)


def _tc4(agg, b):
    return pl.pallas_call(
        _tc4_body,
        grid=(2,),
        in_specs=[_AGG_SPEC, _B_SPEC],
        out_specs=_ROWS_SPEC,
        out_shape=jax.ShapeDtypeStruct((NP, 128), jnp.float32),
    )(agg, b)


def kernel(x, edge_index, W1, b1, W2, b2, Wout, bout):
    src = edge_index[0]
    dst = edge_index[1]

    pad = (N + (jnp.arange(EP - E, dtype=jnp.int32) % (NP - N))).astype(jnp.int32)
    src_p = jnp.concatenate([src, pad]).reshape(ROWS, B)
    dst_p = jnp.concatenate([dst, pad]).reshape(ROWS, B)
    didxN = dst_p + NP
    x_p = jnp.pad(x, ((0, NP - N), (0, 0)))

    deg_k, ep_k = _make_sc_kernels()

    degp = deg_k(src_p, didxN).reshape(NW, 2, NP)
    q1, ns, nd = _tc1(degp, x_p, W1)
    agg1 = ep_k(q1, src_p, dst_p)
    h1, q2 = _tc2(agg1, nd, ns, b1.reshape(1, H), W2)
    agg2 = ep_k(q2, src_p, dst_p)
    z = _tc3(agg2, nd, b2.reshape(1, H), h1, Wout)
    agg3 = ep_k(z, src_p, dst_p)
    outp = _tc4(agg3, bout.reshape(1, OUT))
    return outp[:N]


# speedup vs baseline: 14.8787x; 1.0213x over previous
import functools

import jax
import jax.numpy as jnp
from jax import lax
from jax.experimental import pallas as pl
from jax.experimental.pallas import tpu as pltpu
from jax.experimental.pallas import tpu_sc as plsc

N = 10000
D = 128
E = 320000
H = 128
OUT = 128

NW = 32
B = 128
BPT = 80
ROWS = NW * BPT
EP = ROWS * B

NSUB = 16
RPS = 640
NP = NSUB * RPS
WCH = RPS // B
NPB = NP // 2


def _zero_rows(ref, nrows):
    zeros16 = jnp.zeros((16,), jnp.float32)

    def body(r, _):
        for o in range(8):
            ref[r, pl.ds(o * 16, 16)] = zeros16
        return 0

    lax.fori_loop(0, nrows, body, 0)


def _deg_body(sidx_hbm, didx_hbm, out_hbm, sidx_v, didx_v, acc_v):
    cid = lax.axis_index("c")
    sid = lax.axis_index("s")
    wid = sid * 2 + cid

    def zb(i, _):
        acc_v[pl.ds(i * 16, 16)] = jnp.zeros((16,), jnp.float32)
        return 0

    lax.fori_loop(0, (2 * NP) // 16, zb, 0)

    pltpu.sync_copy(sidx_hbm.at[pl.ds(wid * BPT, BPT)], sidx_v)
    pltpu.sync_copy(didx_hbm.at[pl.ds(wid * BPT, BPT)], didx_v)

    ones16 = jnp.ones((16,), jnp.float32)

    def body(j, _):
        for o in range(8):
            plsc.addupdate_scatter(acc_v, [sidx_v[j, pl.ds(o * 16, 16)]], ones16)
            plsc.addupdate_scatter(acc_v, [didx_v[j, pl.ds(o * 16, 16)]], ones16)
        return 0

    lax.fori_loop(0, BPT, body, 0)
    pltpu.sync_copy(acc_v, out_hbm.at[pl.ds(wid * 2 * NP, 2 * NP)])


IG = 16
NCH = BPT // IG


def _ep_body(q_hbm, sidx_hbm, didx_hbm, out_hbm, sidx_v, di0, di1,
             rv0, rv1, acc_sh, gs0, gs1, isem, zsem):
    cid = lax.axis_index("c")
    sid = lax.axis_index("s")
    wid = sid * 2 + cid
    rows = (rv0, rv1)
    gsems = (gs0, gs1)
    dibufs = (di0, di1)

    pltpu.async_copy(sidx_hbm.at[pl.ds(wid * BPT, BPT)], sidx_v, gs0)
    pltpu.async_copy(didx_hbm.at[pl.ds(wid * BPT, IG)], di0, gs1)

    _zero_rows(rv0, B)
    for k in range(WCH):
        pltpu.async_copy(rv0, acc_sh.at[pl.ds(sid * RPS + k * B, B)], zsem)
    for k in range(WCH):
        pltpu.make_async_copy(rv0, acc_sh.at[pl.ds(0, B)], zsem).wait()
    plsc.subcore_barrier()
    pltpu.make_async_copy(sidx_hbm.at[pl.ds(0, BPT)], sidx_v, gs0).wait()
    pltpu.make_async_copy(didx_hbm.at[pl.ds(0, IG)], di0, gs1).wait()

    def gather(t, b):
        pltpu.async_copy(q_hbm.at[sidx_v.at[t]], rows[b], gsems[b])

    def gwait(b):
        pltpu.make_async_copy(q_hbm.at[sidx_v.at[0]], rows[b], gsems[b]).wait()

    def scatter(r, di, b):
        pltpu.sync_copy(rows[b], acc_sh.at[di.at[r]], add=True)

    def load_chunk(c):
        pltpu.async_copy(didx_hbm.at[pl.ds(wid * BPT + c * IG, IG)],
                         dibufs[c % 2], isem)

    def chunk_wait(c):
        pltpu.make_async_copy(didx_hbm.at[pl.ds(0, IG)], dibufs[c % 2],
                              isem).wait()

    gather(0, 0)
    gather(1, 1)
    load_chunk(1)

    for c in range(NCH):
        di = dibufs[c % 2]
        base = c * IG
        if c >= 1:
            chunk_wait(c)
            if c + 1 < NCH:
                load_chunk(c + 1)

        last = c == NCH - 1
        hi = IG // 2 - 1 if last else IG // 2

        def cpair(i, _, di=di, base=base):
            r = 2 * i
            t = base + r
            gwait(0); scatter(r, di, 0); gather(t + 2, 0)
            gwait(1); scatter(r + 1, di, 1); gather(t + 3, 1)
            return 0

        lax.fori_loop(0, hi, cpair, 0)
        if last:
            gwait(0); scatter(IG - 2, di, 0)
            gwait(1); scatter(IG - 1, di, 1)

    plsc.subcore_barrier()

    for k in range(WCH):
        b = k % 2
        sl = pl.ds(sid * RPS + k * B, B)
        if k >= 2:
            pltpu.make_async_copy(rows[b], out_hbm.at[cid, pl.ds(0, B)],
                                  gsems[b]).wait()
        pltpu.sync_copy(acc_sh.at[sl], rows[b])
        pltpu.async_copy(rows[b], out_hbm.at[cid, sl], gsems[b])
    pltpu.make_async_copy(rv0, out_hbm.at[cid, pl.ds(0, B)], gsems[(WCH - 2) % 2]).wait()
    pltpu.make_async_copy(rv1, out_hbm.at[cid, pl.ds(0, B)], gsems[(WCH - 1) % 2]).wait()


def _make_sc_kernels():
    mesh = plsc.VectorSubcoreMesh(core_axis_name="c", subcore_axis_name="s")
    params = pltpu.CompilerParams(needs_layout_passes=False)
    deg = pl.kernel(
        _deg_body,
        out_type=jax.ShapeDtypeStruct((NW * 2 * NP,), jnp.float32),
        mesh=mesh,
        compiler_params=params,
        scratch_types=[
            pltpu.VMEM((BPT, B), jnp.int32),
            pltpu.VMEM((BPT, B), jnp.int32),
            pltpu.VMEM((2 * NP,), jnp.float32),
        ],
    )
    ep = pl.kernel(
        _ep_body,
        out_type=jax.ShapeDtypeStruct((2, NP, 128), jnp.float32),
        mesh=mesh,
        compiler_params=params,
        scratch_types=(
            [pltpu.VMEM((BPT, B), jnp.int32)]
            + [pltpu.VMEM((IG, B), jnp.int32)] * 2
            + [pltpu.VMEM((B, 128), jnp.float32)] * 2
            + [pltpu.VMEM_SHARED((NP, 128), jnp.float32)]
            + [pltpu.SemaphoreType.DMA] * 4
        ),
    )
    return deg, ep


_PREC = jax.lax.Precision.HIGHEST


def _tc1_body(degp_ref, x_ref, w1_ref, q1_ref, ns_ref, nd_ref):
    deg = jnp.sum(degp_ref[...], axis=0)
    ns = lax.rsqrt(jnp.maximum(deg[0], 1.0))[:, None]
    nd = lax.rsqrt(jnp.maximum(deg[1], 1.0))[:, None]
    ns_ref[...] = ns
    nd_ref[...] = nd
    q1_ref[...] = jnp.dot(x_ref[...] * ns, w1_ref[...],
                          preferred_element_type=jnp.float32, precision=_PREC)


def _tc2_body(agg_ref, nd_ref, ns_ref, b_ref, w_ref, h_ref, q_ref):
    s = agg_ref[0] + agg_ref[1]
    h = jnp.maximum(s * nd_ref[...] + b_ref[...], 0.0)
    h_ref[...] = h
    q_ref[...] = jnp.dot(h * ns_ref[...], w_ref[...],
                         preferred_element_type=jnp.float32, precision=_PREC)


def _tc3_body(agg_ref, nd_ref, b_ref, h1_ref, wout_ref, z_ref):
    s = agg_ref[0] + agg_ref[1]
    h2 = jnp.maximum(s * nd_ref[...] + b_ref[...], 0.0)
    z_ref[...] = (
        jnp.dot(h1_ref[...], wout_ref[:H], preferred_element_type=jnp.float32,
                precision=_PREC)
        + jnp.dot(h2, wout_ref[H:], preferred_element_type=jnp.float32,
                  precision=_PREC)
    )


def _tc4_body(agg_ref, b_ref, out_ref):
    out_ref[...] = agg_ref[0] + agg_ref[1] + b_ref[...]


def _row_blk(i):
    return (i, 0)


def _agg_blk(i):
    return (0, i, 0)


def _full_blk(i):
    return (0, 0)


_ROWS_SPEC = pl.BlockSpec((NPB, 128), _row_blk)
_COL_SPEC = pl.BlockSpec((NPB, 1), _row_blk)
_AGG_SPEC = pl.BlockSpec((2, NPB, 128), _agg_blk)
_B_SPEC = pl.BlockSpec((1, 128), _full_blk)


def _tc1(degp, x_p, W1):
    return pl.pallas_call(
        _tc1_body,
        grid=(2,),
        in_specs=[
            pl.BlockSpec((NW, 2, NPB), lambda i: (0, 0, i)),
            _ROWS_SPEC,
            pl.BlockSpec((128, 128), _full_blk),
        ],
        out_specs=[_ROWS_SPEC, _COL_SPEC, _COL_SPEC],
        out_shape=[
            jax.ShapeDtypeStruct((NP, 128), jnp.float32),
            jax.ShapeDtypeStruct((NP, 1), jnp.float32),
            jax.ShapeDtypeStruct((NP, 1), jnp.float32),
        ],
    )(degp, x_p, W1)


def _tc2(agg, nd, ns, b, W):
    return pl.pallas_call(
        _tc2_body,
        grid=(2,),
        in_specs=[_AGG_SPEC, _COL_SPEC, _COL_SPEC, _B_SPEC,
                  pl.BlockSpec((128, 128), _full_blk)],
        out_specs=[_ROWS_SPEC, _ROWS_SPEC],
        out_shape=[
            jax.ShapeDtypeStruct((NP, 128), jnp.float32),
            jax.ShapeDtypeStruct((NP, 128), jnp.float32),
        ],
    )(agg, nd, ns, b, W)


def _tc3(agg, nd, b, h1, Wout):
    return pl.pallas_call(
        _tc3_body,
        grid=(2,),
        in_specs=[_AGG_SPEC, _COL_SPEC, _B_SPEC, _ROWS_SPEC,
                  pl.BlockSpec((2 * H, 128), _full_blk)],
        out_specs=_ROWS_SPEC,
        out_shape=jax.ShapeDtypeStruct((NP, 128), jnp.float32),
    )(agg, nd, b, h1, Wout)


def _tc4(agg, b):
    return pl.pallas_call(
        _tc4_body,
        grid=(2,),
        in_specs=[_AGG_SPEC, _B_SPEC],
        out_specs=_ROWS_SPEC,
        out_shape=jax.ShapeDtypeStruct((NP, 128), jnp.float32),
    )(agg, b)


def kernel(x, edge_index, W1, b1, W2, b2, Wout, bout):
    src = edge_index[0]
    dst = edge_index[1]

    pad = (N + (jnp.arange(EP - E, dtype=jnp.int32) % (NP - N))).astype(jnp.int32)
    src_p = jnp.concatenate([src, pad]).reshape(ROWS, B)
    dst_p = jnp.concatenate([dst, pad]).reshape(ROWS, B)
    didxN = dst_p + NP
    x_p = jnp.pad(x, ((0, NP - N), (0, 0)))

    deg_k, ep_k = _make_sc_kernels()

    degp = deg_k(src_p, didxN).reshape(NW, 2, NP)
    q1, ns, nd = _tc1(degp, x_p, W1)
    agg1 = ep_k(q1, src_p, dst_p)
    h1, q2 = _tc2(agg1, nd, ns, b1.reshape(1, H), W2)
    agg2 = ep_k(q2, src_p, dst_p)
    z = _tc3(agg2, nd, b2.reshape(1, H), h1, Wout)
    agg3 = ep_k(z, src_p, dst_p)
    outp = _tc4(agg3, bout.reshape(1, OUT))
    return outp[:N]


# speedup vs baseline: 15.1136x; 1.0158x over previous
import functools

import jax
import jax.numpy as jnp
from jax import lax
from jax.experimental import pallas as pl
from jax.experimental.pallas import tpu as pltpu
from jax.experimental.pallas import tpu_sc as plsc

N = 10000
D = 128
E = 320000
H = 128
OUT = 128

NW = 32
B = 128
BPT = 80
ROWS = NW * BPT
EP = ROWS * B

NSUB = 16
RPS = 640
NP = NSUB * RPS
WCH = RPS // B
NPB = NP // 2


def _zero_rows(ref, nrows):
    zeros16 = jnp.zeros((16,), jnp.float32)

    def body(r, _):
        for o in range(8):
            ref[r, pl.ds(o * 16, 16)] = zeros16
        return 0

    lax.fori_loop(0, nrows, body, 0)


def _deg_body(sidx_hbm, didx_hbm, out_hbm, sidx_v, didx_v, acc_v):
    cid = lax.axis_index("c")
    sid = lax.axis_index("s")
    wid = sid * 2 + cid

    def zb(i, _):
        acc_v[pl.ds(i * 16, 16)] = jnp.zeros((16,), jnp.float32)
        return 0

    lax.fori_loop(0, (2 * NP) // 16, zb, 0)

    pltpu.sync_copy(sidx_hbm.at[pl.ds(wid * BPT, BPT)], sidx_v)
    pltpu.sync_copy(didx_hbm.at[pl.ds(wid * BPT, BPT)], didx_v)

    ones16 = jnp.ones((16,), jnp.float32)

    npv = jnp.full((16,), NP, jnp.int32)

    def body(j, _):
        for o in range(8):
            plsc.addupdate_scatter(acc_v, [sidx_v[j, pl.ds(o * 16, 16)]], ones16)
            plsc.addupdate_scatter(acc_v, [didx_v[j, pl.ds(o * 16, 16)] + npv],
                                   ones16)
        return 0

    lax.fori_loop(0, BPT, body, 0)
    pltpu.sync_copy(acc_v, out_hbm.at[pl.ds(wid * 2 * NP, 2 * NP)])


IG = 16
NCH = BPT // IG


def _ep_body(q_hbm, sidx_hbm, didx_hbm, out_hbm, sidx_v, di0, di1,
             rv0, rv1, acc_sh, gs0, gs1, isem, zsem):
    cid = lax.axis_index("c")
    sid = lax.axis_index("s")
    wid = sid * 2 + cid
    rows = (rv0, rv1)
    gsems = (gs0, gs1)
    dibufs = (di0, di1)

    pltpu.async_copy(sidx_hbm.at[pl.ds(wid * BPT, BPT)], sidx_v, gs0)
    pltpu.async_copy(didx_hbm.at[pl.ds(wid * BPT, IG)], di0, gs1)

    _zero_rows(rv0, B)
    for k in range(WCH):
        pltpu.async_copy(rv0, acc_sh.at[pl.ds(sid * RPS + k * B, B)], zsem)
    for k in range(WCH):
        pltpu.make_async_copy(rv0, acc_sh.at[pl.ds(0, B)], zsem).wait()
    plsc.subcore_barrier()
    pltpu.make_async_copy(sidx_hbm.at[pl.ds(0, BPT)], sidx_v, gs0).wait()
    pltpu.make_async_copy(didx_hbm.at[pl.ds(0, IG)], di0, gs1).wait()

    def gather(t, b):
        pltpu.async_copy(q_hbm.at[sidx_v.at[t]], rows[b], gsems[b])

    def gwait(b):
        pltpu.make_async_copy(q_hbm.at[sidx_v.at[0]], rows[b], gsems[b]).wait()

    def scatter(r, di, b):
        pltpu.sync_copy(rows[b], acc_sh.at[di.at[r]], add=True)

    def load_chunk(c):
        pltpu.async_copy(didx_hbm.at[pl.ds(wid * BPT + c * IG, IG)],
                         dibufs[c % 2], isem)

    def chunk_wait(c):
        pltpu.make_async_copy(didx_hbm.at[pl.ds(0, IG)], dibufs[c % 2],
                              isem).wait()

    gather(0, 0)
    gather(1, 1)
    load_chunk(1)

    for c in range(NCH):
        di = dibufs[c % 2]
        base = c * IG
        if c >= 1:
            chunk_wait(c)
            if c + 1 < NCH:
                load_chunk(c + 1)

        last = c == NCH - 1
        hi = IG // 2 - 1 if last else IG // 2

        def cpair(i, _, di=di, base=base):
            r = 2 * i
            t = base + r
            gwait(0); scatter(r, di, 0); gather(t + 2, 0)
            gwait(1); scatter(r + 1, di, 1); gather(t + 3, 1)
            return 0

        lax.fori_loop(0, hi, cpair, 0)
        if last:
            gwait(0); scatter(IG - 2, di, 0)
            gwait(1); scatter(IG - 1, di, 1)

    plsc.subcore_barrier()

    for k in range(WCH):
        b = k % 2
        sl = pl.ds(sid * RPS + k * B, B)
        if k >= 2:
            pltpu.make_async_copy(rows[b], out_hbm.at[cid, pl.ds(0, B)],
                                  gsems[b]).wait()
        pltpu.sync_copy(acc_sh.at[sl], rows[b])
        pltpu.async_copy(rows[b], out_hbm.at[cid, sl], gsems[b])
    pltpu.make_async_copy(rv0, out_hbm.at[cid, pl.ds(0, B)], gsems[(WCH - 2) % 2]).wait()
    pltpu.make_async_copy(rv1, out_hbm.at[cid, pl.ds(0, B)], gsems[(WCH - 1) % 2]).wait()


def _make_sc_kernels():
    mesh = plsc.VectorSubcoreMesh(core_axis_name="c", subcore_axis_name="s")
    params = pltpu.CompilerParams(needs_layout_passes=False)
    deg = pl.kernel(
        _deg_body,
        out_type=jax.ShapeDtypeStruct((NW * 2 * NP,), jnp.float32),
        mesh=mesh,
        compiler_params=params,
        scratch_types=[
            pltpu.VMEM((BPT, B), jnp.int32),
            pltpu.VMEM((BPT, B), jnp.int32),
            pltpu.VMEM((2 * NP,), jnp.float32),
        ],
    )
    ep = pl.kernel(
        _ep_body,
        out_type=jax.ShapeDtypeStruct((2, NP, 128), jnp.float32),
        mesh=mesh,
        compiler_params=params,
        scratch_types=(
            [pltpu.VMEM((BPT, B), jnp.int32)]
            + [pltpu.VMEM((IG, B), jnp.int32)] * 2
            + [pltpu.VMEM((B, 128), jnp.float32)] * 2
            + [pltpu.VMEM_SHARED((NP, 128), jnp.float32)]
            + [pltpu.SemaphoreType.DMA] * 4
        ),
    )
    return deg, ep


_PREC = jax.lax.Precision.HIGHEST


def _tc0_body(x_ref, w1_ref, u1_ref):
    u1_ref[...] = jnp.dot(x_ref[...], w1_ref[...],
                          preferred_element_type=jnp.float32, precision=_PREC)


def _tc1_body(degp_ref, u1_ref, q1_ref, ns_ref, nd_ref):
    deg = jnp.sum(degp_ref[...], axis=0)
    ns = lax.rsqrt(jnp.maximum(deg[0], 1.0))[:, None]
    nd = lax.rsqrt(jnp.maximum(deg[1], 1.0))[:, None]
    ns_ref[...] = ns
    nd_ref[...] = nd
    q1_ref[...] = u1_ref[...] * ns


def _tc2_body(agg_ref, nd_ref, ns_ref, b_ref, w_ref, h_ref, q_ref):
    s = agg_ref[0] + agg_ref[1]
    h = jnp.maximum(s * nd_ref[...] + b_ref[...], 0.0)
    h_ref[...] = h
    q_ref[...] = jnp.dot(h * ns_ref[...], w_ref[...],
                         preferred_element_type=jnp.float32, precision=_PREC)


def _tc3_body(agg_ref, nd_ref, b_ref, h1_ref, wout_ref, z_ref):
    s = agg_ref[0] + agg_ref[1]
    h2 = jnp.maximum(s * nd_ref[...] + b_ref[...], 0.0)
    z_ref[...] = (
        jnp.dot(h1_ref[...], wout_ref[:H], preferred_element_type=jnp.float32,
                precision=_PREC)
        + jnp.dot(h2, wout_ref[H:], preferred_element_type=jnp.float32,
                  precision=_PREC)
    )


def _tc4_body(agg_ref, b_ref, out_ref):
    out_ref[...] = agg_ref[0] + agg_ref[1] + b_ref[...]


def _row_blk(i):
    return (i, 0)


def _agg_blk(i):
    return (0, i, 0)


def _full_blk(i):
    return (0, 0)


_ROWS_SPEC = pl.BlockSpec((NPB, 128), _row_blk)
_COL_SPEC = pl.BlockSpec((NPB, 1), _row_blk)
_AGG_SPEC = pl.BlockSpec((2, NPB, 128), _agg_blk)
_B_SPEC = pl.BlockSpec((1, 128), _full_blk)


def _tc0(x_p, W1):
    return pl.pallas_call(
        _tc0_body,
        grid=(2,),
        in_specs=[_ROWS_SPEC, pl.BlockSpec((128, 128), _full_blk)],
        out_specs=_ROWS_SPEC,
        out_shape=jax.ShapeDtypeStruct((NP, 128), jnp.float32),
    )(x_p, W1)


def _tc1(degp, u1):
    return pl.pallas_call(
        _tc1_body,
        grid=(2,),
        in_specs=[
            pl.BlockSpec((NW, 2, NPB), lambda i: (0, 0, i)),
            _ROWS_SPEC,
        ],
        out_specs=[_ROWS_SPEC, _COL_SPEC, _COL_SPEC],
        out_shape=[
            jax.ShapeDtypeStruct((NP, 128), jnp.float32),
            jax.ShapeDtypeStruct((NP, 1), jnp.float32),
            jax.ShapeDtypeStruct((NP, 1), jnp.float32),
        ],
    )(degp, u1)


def _tc2(agg, nd, ns, b, W):
    return pl.pallas_call(
        _tc2_body,
        grid=(2,),
        in_specs=[_AGG_SPEC, _COL_SPEC, _COL_SPEC, _B_SPEC,
                  pl.BlockSpec((128, 128), _full_blk)],
        out_specs=[_ROWS_SPEC, _ROWS_SPEC],
        out_shape=[
            jax.ShapeDtypeStruct((NP, 128), jnp.float32),
            jax.ShapeDtypeStruct((NP, 128), jnp.float32),
        ],
    )(agg, nd, ns, b, W)


def _tc3(agg, nd, b, h1, Wout):
    return pl.pallas_call(
        _tc3_body,
        grid=(2,),
        in_specs=[_AGG_SPEC, _COL_SPEC, _B_SPEC, _ROWS_SPEC,
                  pl.BlockSpec((2 * H, 128), _full_blk)],
        out_specs=_ROWS_SPEC,
        out_shape=jax.ShapeDtypeStruct((NP, 128), jnp.float32),
    )(agg, nd, b, h1, Wout)


def _tc4(agg, b):
    return pl.pallas_call(
        _tc4_body,
        grid=(5,),
        in_specs=[pl.BlockSpec((2, N // 5, 128), lambda i: (0, i, 0)), _B_SPEC],
        out_specs=pl.BlockSpec((N // 5, 128), _row_blk),
        out_shape=jax.ShapeDtypeStruct((N, 128), jnp.float32),
    )(agg, b)


def kernel(x, edge_index, W1, b1, W2, b2, Wout, bout):
    src = edge_index[0]
    dst = edge_index[1]

    pad = (N + (jnp.arange(EP - E, dtype=jnp.int32) % (NP - N))).astype(jnp.int32)
    src_p = jnp.concatenate([src, pad]).reshape(ROWS, B)
    dst_p = jnp.concatenate([dst, pad]).reshape(ROWS, B)
    x_p = jnp.pad(x, ((0, NP - N), (0, 0)))

    deg_k, ep_k = _make_sc_kernels()

    u1 = _tc0(x_p, W1)
    degp = deg_k(src_p, dst_p).reshape(NW, 2, NP)
    q1, ns, nd = _tc1(degp, u1)
    agg1 = ep_k(q1, src_p, dst_p)
    h1, q2 = _tc2(agg1, nd, ns, b1.reshape(1, H), W2)
    agg2 = ep_k(q2, src_p, dst_p)
    z = _tc3(agg2, nd, b2.reshape(1, H), h1, Wout)
    agg3 = ep_k(z, src_p, dst_p)
    return _tc4(agg3, bout.reshape(1, OUT))


# speedup vs baseline: 15.3020x; 1.0125x over previous
import functools

import jax
import jax.numpy as jnp
from jax import lax
from jax.experimental import pallas as pl
from jax.experimental.pallas import tpu as pltpu
from jax.experimental.pallas import tpu_sc as plsc

N = 10000
D = 128
E = 320000
H = 128
OUT = 128

NW = 32
B = 128
BPT = 80
ROWS = NW * BPT
EP = ROWS * B

NSUB = 16
RPS = 640
NP = NSUB * RPS
WCH = RPS // B
NPB = NP // 2


def _zero_rows(ref, nrows):
    zeros16 = jnp.zeros((16,), jnp.float32)

    def body(r, _):
        for o in range(8):
            ref[r, pl.ds(o * 16, 16)] = zeros16
        return 0

    lax.fori_loop(0, nrows, body, 0)


def _deg_body(sidx_hbm, didx_hbm, out_hbm, sidx_v, didx_v, acc_v):
    cid = lax.axis_index("c")
    sid = lax.axis_index("s")
    wid = sid * 2 + cid

    def zb(i, _):
        acc_v[pl.ds(i * 16, 16)] = jnp.zeros((16,), jnp.float32)
        return 0

    lax.fori_loop(0, (2 * NP) // 16, zb, 0)

    pltpu.sync_copy(sidx_hbm.at[pl.ds(wid * BPT, BPT)], sidx_v)
    pltpu.sync_copy(didx_hbm.at[pl.ds(wid * BPT, BPT)], didx_v)

    ones16 = jnp.ones((16,), jnp.float32)

    npv = jnp.full((16,), NP, jnp.int32)

    def body(j, _):
        for o in range(8):
            plsc.addupdate_scatter(acc_v, [sidx_v[j, pl.ds(o * 16, 16)]], ones16)
            plsc.addupdate_scatter(acc_v, [didx_v[j, pl.ds(o * 16, 16)] + npv],
                                   ones16)
        return 0

    lax.fori_loop(0, BPT, body, 0)
    pltpu.sync_copy(acc_v, out_hbm.at[pl.ds(wid * 2 * NP, 2 * NP)])


IG = 16
NCH = BPT // IG


def _ep_body(q_hbm, sidx_hbm, didx_hbm, out_hbm, sidx_v, di0, di1,
             rv0, rv1, acc_sh, gs0, gs1, isem, zsem):
    cid = lax.axis_index("c")
    sid = lax.axis_index("s")
    wid = sid * 2 + cid
    rows = (rv0, rv1)
    gsems = (gs0, gs1)
    dibufs = (di0, di1)

    pltpu.async_copy(sidx_hbm.at[pl.ds(wid * BPT, BPT)], sidx_v, gs0)
    pltpu.async_copy(didx_hbm.at[pl.ds(wid * BPT, IG)], di0, gs1)

    _zero_rows(rv0, B)
    for k in range(WCH):
        pltpu.async_copy(rv0, acc_sh.at[pl.ds(sid * RPS + k * B, B)], zsem)
    for k in range(WCH):
        pltpu.make_async_copy(rv0, acc_sh.at[pl.ds(0, B)], zsem).wait()
    plsc.subcore_barrier()
    pltpu.make_async_copy(sidx_hbm.at[pl.ds(0, BPT)], sidx_v, gs0).wait()
    pltpu.make_async_copy(didx_hbm.at[pl.ds(0, IG)], di0, gs1).wait()

    def gather(t, b):
        pltpu.async_copy(q_hbm.at[sidx_v.at[t]], rows[b], gsems[b])

    def gwait(b):
        pltpu.make_async_copy(q_hbm.at[sidx_v.at[0]], rows[b], gsems[b]).wait()

    def scatter(r, di, b):
        pltpu.sync_copy(rows[b], acc_sh.at[di.at[r]], add=True)

    def load_chunk(c):
        pltpu.async_copy(didx_hbm.at[pl.ds(wid * BPT + c * IG, IG)],
                         dibufs[c % 2], isem)

    def chunk_wait(c):
        pltpu.make_async_copy(didx_hbm.at[pl.ds(0, IG)], dibufs[c % 2],
                              isem).wait()

    gather(0, 0)
    gather(1, 1)
    load_chunk(1)

    for c in range(NCH):
        di = dibufs[c % 2]
        base = c * IG
        if c >= 1:
            chunk_wait(c)
            if c + 1 < NCH:
                load_chunk(c + 1)

        last = c == NCH - 1
        hi = IG // 2 - 1 if last else IG // 2

        def cpair(i, _, di=di, base=base):
            r = 2 * i
            t = base + r
            gwait(0); scatter(r, di, 0); gather(t + 2, 0)
            gwait(1); scatter(r + 1, di, 1); gather(t + 3, 1)
            return 0

        lax.fori_loop(0, hi, cpair, 0)
        if last:
            gwait(0); scatter(IG - 2, di, 0)
            gwait(1); scatter(IG - 1, di, 1)

    plsc.subcore_barrier()

    for k in range(WCH):
        b = k % 2
        sl = pl.ds(sid * RPS + k * B, B)
        if k >= 2:
            pltpu.make_async_copy(rows[b], out_hbm.at[cid, pl.ds(0, B)],
                                  gsems[b]).wait()
        pltpu.sync_copy(acc_sh.at[sl], rows[b])
        pltpu.async_copy(rows[b], out_hbm.at[cid, sl], gsems[b])
    pltpu.make_async_copy(rv0, out_hbm.at[cid, pl.ds(0, B)], gsems[(WCH - 2) % 2]).wait()
    pltpu.make_async_copy(rv1, out_hbm.at[cid, pl.ds(0, B)], gsems[(WCH - 1) % 2]).wait()


def _make_sc_kernels():
    mesh = plsc.VectorSubcoreMesh(core_axis_name="c", subcore_axis_name="s")
    params = pltpu.CompilerParams(needs_layout_passes=False)
    deg = pl.kernel(
        _deg_body,
        out_type=jax.ShapeDtypeStruct((NW * 2 * NP,), jnp.float32),
        mesh=mesh,
        compiler_params=params,
        scratch_types=[
            pltpu.VMEM((BPT, B), jnp.int32),
            pltpu.VMEM((BPT, B), jnp.int32),
            pltpu.VMEM((2 * NP,), jnp.float32),
        ],
    )
    ep = pl.kernel(
        _ep_body,
        out_type=jax.ShapeDtypeStruct((2, NP, 128), jnp.float32),
        mesh=mesh,
        compiler_params=params,
        scratch_types=(
            [pltpu.VMEM((BPT, B), jnp.int32)]
            + [pltpu.VMEM((IG, B), jnp.int32)] * 2
            + [pltpu.VMEM((B, 128), jnp.float32)] * 2
            + [pltpu.VMEM_SHARED((NP, 128), jnp.float32)]
            + [pltpu.SemaphoreType.DMA] * 4
        ),
    )
    return deg, ep


_PREC = None


def _tc0_body(x_ref, w1_ref, u1_ref):
    u1_ref[...] = jnp.dot(x_ref[...], w1_ref[...],
                          preferred_element_type=jnp.float32, precision=_PREC)


def _tc1_body(degp_ref, u1_ref, q1_ref, ns_ref, nd_ref):
    deg = jnp.sum(degp_ref[...], axis=0)
    ns = lax.rsqrt(jnp.maximum(deg[0], 1.0))[:, None]
    nd = lax.rsqrt(jnp.maximum(deg[1], 1.0))[:, None]
    ns_ref[...] = ns
    nd_ref[...] = nd
    q1_ref[...] = u1_ref[...] * ns


def _tc2_body(agg_ref, nd_ref, ns_ref, b_ref, w_ref, h_ref, q_ref):
    s = agg_ref[0] + agg_ref[1]
    h = jnp.maximum(s * nd_ref[...] + b_ref[...], 0.0)
    h_ref[...] = h
    q_ref[...] = jnp.dot(h * ns_ref[...], w_ref[...],
                         preferred_element_type=jnp.float32, precision=_PREC)


def _tc3_body(agg_ref, nd_ref, b_ref, h1_ref, wout_ref, z_ref):
    s = agg_ref[0] + agg_ref[1]
    h2 = jnp.maximum(s * nd_ref[...] + b_ref[...], 0.0)
    z_ref[...] = (
        jnp.dot(h1_ref[...], wout_ref[:H], preferred_element_type=jnp.float32,
                precision=_PREC)
        + jnp.dot(h2, wout_ref[H:], preferred_element_type=jnp.float32,
                  precision=_PREC)
    )


def _tc4_body(agg_ref, b_ref, out_ref):
    out_ref[...] = agg_ref[0] + agg_ref[1] + b_ref[...]


def _row_blk(i):
    return (i, 0)


def _agg_blk(i):
    return (0, i, 0)


def _full_blk(i):
    return (0, 0)


_ROWS_SPEC = pl.BlockSpec((NPB, 128), _row_blk)
_COL_SPEC = pl.BlockSpec((NPB, 1), _row_blk)
_AGG_SPEC = pl.BlockSpec((2, NPB, 128), _agg_blk)
_B_SPEC = pl.BlockSpec((1, 128), _full_blk)


def _tc0(x_p, W1):
    return pl.pallas_call(
        _tc0_body,
        grid=(2,),
        in_specs=[_ROWS_SPEC, pl.BlockSpec((128, 128), _full_blk)],
        out_specs=_ROWS_SPEC,
        out_shape=jax.ShapeDtypeStruct((NP, 128), jnp.float32),
    )(x_p, W1)


def _tc1(degp, u1):
    return pl.pallas_call(
        _tc1_body,
        grid=(2,),
        in_specs=[
            pl.BlockSpec((NW, 2, NPB), lambda i: (0, 0, i)),
            _ROWS_SPEC,
        ],
        out_specs=[_ROWS_SPEC, _COL_SPEC, _COL_SPEC],
        out_shape=[
            jax.ShapeDtypeStruct((NP, 128), jnp.float32),
            jax.ShapeDtypeStruct((NP, 1), jnp.float32),
            jax.ShapeDtypeStruct((NP, 1), jnp.float32),
        ],
    )(degp, u1)


def _tc2(agg, nd, ns, b, W):
    return pl.pallas_call(
        _tc2_body,
        grid=(2,),
        in_specs=[_AGG_SPEC, _COL_SPEC, _COL_SPEC, _B_SPEC,
                  pl.BlockSpec((128, 128), _full_blk)],
        out_specs=[_ROWS_SPEC, _ROWS_SPEC],
        out_shape=[
            jax.ShapeDtypeStruct((NP, 128), jnp.float32),
            jax.ShapeDtypeStruct((NP, 128), jnp.float32),
        ],
    )(agg, nd, ns, b, W)


def _tc3(agg, nd, b, h1, Wout):
    return pl.pallas_call(
        _tc3_body,
        grid=(2,),
        in_specs=[_AGG_SPEC, _COL_SPEC, _B_SPEC, _ROWS_SPEC,
                  pl.BlockSpec((2 * H, 128), _full_blk)],
        out_specs=_ROWS_SPEC,
        out_shape=jax.ShapeDtypeStruct((NP, 128), jnp.float32),
    )(agg, nd, b, h1, Wout)


def _tc4(agg, b):
    return pl.pallas_call(
        _tc4_body,
        grid=(5,),
        in_specs=[pl.BlockSpec((2, N // 5, 128), lambda i: (0, i, 0)), _B_SPEC],
        out_specs=pl.BlockSpec((N // 5, 128), _row_blk),
        out_shape=jax.ShapeDtypeStruct((N, 128), jnp.float32),
    )(agg, b)


def kernel(x, edge_index, W1, b1, W2, b2, Wout, bout):
    src = edge_index[0]
    dst = edge_index[1]

    pad = (N + (jnp.arange(EP - E, dtype=jnp.int32) % (NP - N))).astype(jnp.int32)
    src_p = jnp.concatenate([src, pad]).reshape(ROWS, B)
    dst_p = jnp.concatenate([dst, pad]).reshape(ROWS, B)
    x_p = jnp.pad(x, ((0, NP - N), (0, 0)))

    deg_k, ep_k = _make_sc_kernels()

    u1 = _tc0(x_p, W1)
    degp = deg_k(src_p, dst_p).reshape(NW, 2, NP)
    q1, ns, nd = _tc1(degp, u1)
    agg1 = ep_k(q1, src_p, dst_p)
    h1, q2 = _tc2(agg1, nd, ns, b1.reshape(1, H), W2)
    agg2 = ep_k(q2, src_p, dst_p)
    z = _tc3(agg2, nd, b2.reshape(1, H), h1, Wout)
    agg3 = ep_k(z, src_p, dst_p)
    return _tc4(agg3, bout.reshape(1, OUT))


# speedup vs baseline: 15.4600x; 1.0103x over previous
import functools

import jax
import jax.numpy as jnp
from jax import lax
from jax.experimental import pallas as pl
from jax.experimental.pallas import tpu as pltpu
from jax.experimental.pallas import tpu_sc as plsc

N = 10000
D = 128
E = 320000
H = 128
OUT = 128

NW = 32
B = 128
BPT = 80
ROWS = NW * BPT
EP = ROWS * B

NSUB = 16
RPS = 640
NP = NSUB * RPS
WCH = RPS // B
NPB = NP // 2


def _zero_rows(ref, nrows):
    zeros16 = jnp.zeros((16,), jnp.float32)

    def body(r, _):
        for o in range(8):
            ref[r, pl.ds(o * 16, 16)] = zeros16
        return 0

    lax.fori_loop(0, nrows, body, 0)


def _deg_body(sidx_hbm, didx_hbm, out_hbm, sidx_v, didx_v, acc_v):
    cid = lax.axis_index("c")
    sid = lax.axis_index("s")
    wid = sid * 2 + cid

    def zb(i, _):
        for u in range(8):
            acc_v[pl.ds(i * 128 + u * 16, 16)] = jnp.zeros((16,), jnp.float32)
        return 0

    lax.fori_loop(0, (2 * NP) // 128, zb, 0)

    pltpu.sync_copy(sidx_hbm.at[pl.ds(wid * BPT, BPT)], sidx_v)
    pltpu.sync_copy(didx_hbm.at[pl.ds(wid * BPT, BPT)], didx_v)

    ones16 = jnp.ones((16,), jnp.float32)

    npv = jnp.full((16,), NP, jnp.int32)

    def body(j, _):
        for o in range(8):
            plsc.addupdate_scatter(acc_v, [sidx_v[j, pl.ds(o * 16, 16)]], ones16)
            plsc.addupdate_scatter(acc_v, [didx_v[j, pl.ds(o * 16, 16)] + npv],
                                   ones16)
        return 0

    lax.fori_loop(0, BPT, body, 0)
    pltpu.sync_copy(acc_v, out_hbm.at[pl.ds(wid * 2 * NP, 2 * NP)])


IG = 16
NCH = BPT // IG


def _ep_body(q_hbm, sidx_hbm, didx_hbm, out_hbm, sidx_v, di0, di1,
             rv0, rv1, acc_sh, gs0, gs1, isem, zsem):
    cid = lax.axis_index("c")
    sid = lax.axis_index("s")
    wid = sid * 2 + cid
    rows = (rv0, rv1)
    gsems = (gs0, gs1)
    dibufs = (di0, di1)

    pltpu.async_copy(sidx_hbm.at[pl.ds(wid * BPT, BPT)], sidx_v, gs0)
    pltpu.async_copy(didx_hbm.at[pl.ds(wid * BPT, IG)], di0, gs1)

    _zero_rows(rv0, B)
    for k in range(WCH):
        pltpu.async_copy(rv0, acc_sh.at[pl.ds(sid * RPS + k * B, B)], zsem)
    for k in range(WCH):
        pltpu.make_async_copy(rv0, acc_sh.at[pl.ds(0, B)], zsem).wait()
    plsc.subcore_barrier()
    pltpu.make_async_copy(sidx_hbm.at[pl.ds(0, BPT)], sidx_v, gs0).wait()
    pltpu.make_async_copy(didx_hbm.at[pl.ds(0, IG)], di0, gs1).wait()

    def gather(t, b):
        pltpu.async_copy(q_hbm.at[sidx_v.at[t]], rows[b], gsems[b])

    def gwait(b):
        pltpu.make_async_copy(q_hbm.at[sidx_v.at[0]], rows[b], gsems[b]).wait()

    def scatter(r, di, b):
        pltpu.sync_copy(rows[b], acc_sh.at[di.at[r]], add=True)

    def load_chunk(c):
        pltpu.async_copy(didx_hbm.at[pl.ds(wid * BPT + c * IG, IG)],
                         dibufs[c % 2], isem)

    def chunk_wait(c):
        pltpu.make_async_copy(didx_hbm.at[pl.ds(0, IG)], dibufs[c % 2],
                              isem).wait()

    gather(0, 0)
    gather(1, 1)
    load_chunk(1)

    for c in range(NCH):
        di = dibufs[c % 2]
        base = c * IG
        if c >= 1:
            chunk_wait(c)
            if c + 1 < NCH:
                load_chunk(c + 1)

        last = c == NCH - 1
        hi = IG // 2 - 1 if last else IG // 2

        def cpair(i, _, di=di, base=base):
            r = 2 * i
            t = base + r
            gwait(0); scatter(r, di, 0); gather(t + 2, 0)
            gwait(1); scatter(r + 1, di, 1); gather(t + 3, 1)
            return 0

        lax.fori_loop(0, hi, cpair, 0)
        if last:
            gwait(0); scatter(IG - 2, di, 0)
            gwait(1); scatter(IG - 1, di, 1)

    plsc.subcore_barrier()

    for k in range(WCH):
        b = k % 2
        sl = pl.ds(sid * RPS + k * B, B)
        if k >= 2:
            pltpu.make_async_copy(rows[b], out_hbm.at[cid, pl.ds(0, B)],
                                  gsems[b]).wait()
        pltpu.sync_copy(acc_sh.at[sl], rows[b])
        pltpu.async_copy(rows[b], out_hbm.at[cid, sl], gsems[b])
    pltpu.make_async_copy(rv0, out_hbm.at[cid, pl.ds(0, B)], gsems[(WCH - 2) % 2]).wait()
    pltpu.make_async_copy(rv1, out_hbm.at[cid, pl.ds(0, B)], gsems[(WCH - 1) % 2]).wait()


def _make_sc_kernels():
    mesh = plsc.VectorSubcoreMesh(core_axis_name="c", subcore_axis_name="s")
    params = pltpu.CompilerParams(needs_layout_passes=False)
    deg = pl.kernel(
        _deg_body,
        out_type=jax.ShapeDtypeStruct((NW * 2 * NP,), jnp.float32),
        mesh=mesh,
        compiler_params=params,
        scratch_types=[
            pltpu.VMEM((BPT, B), jnp.int32),
            pltpu.VMEM((BPT, B), jnp.int32),
            pltpu.VMEM((2 * NP,), jnp.float32),
        ],
    )
    ep = pl.kernel(
        _ep_body,
        out_type=jax.ShapeDtypeStruct((2, NP, 128), jnp.float32),
        mesh=mesh,
        compiler_params=params,
        scratch_types=(
            [pltpu.VMEM((BPT, B), jnp.int32)]
            + [pltpu.VMEM((IG, B), jnp.int32)] * 2
            + [pltpu.VMEM((B, 128), jnp.float32)] * 2
            + [pltpu.VMEM_SHARED((NP, 128), jnp.float32)]
            + [pltpu.SemaphoreType.DMA] * 4
        ),
    )
    return deg, ep


_PREC = None


def _tc0_body(x_ref, w1_ref, u1_ref):
    u1_ref[...] = jnp.dot(x_ref[...], w1_ref[...],
                          preferred_element_type=jnp.float32, precision=_PREC)


def _tc1_body(degp_ref, u1_ref, q1_ref, ns_ref, nd_ref):
    deg = jnp.sum(degp_ref[...], axis=0)
    ns = lax.rsqrt(jnp.maximum(deg[0], 1.0))[:, None]
    nd = lax.rsqrt(jnp.maximum(deg[1], 1.0))[:, None]
    ns_ref[...] = ns
    nd_ref[...] = nd
    q1_ref[...] = u1_ref[...] * ns


def _tc2_body(agg_ref, nd_ref, ns_ref, b_ref, w_ref, h_ref, q_ref):
    s = agg_ref[0] + agg_ref[1]
    h = jnp.maximum(s * nd_ref[...] + b_ref[...], 0.0)
    h_ref[...] = h
    q_ref[...] = jnp.dot(h * ns_ref[...], w_ref[...],
                         preferred_element_type=jnp.float32, precision=_PREC)


def _tc3_body(agg_ref, nd_ref, b_ref, h1_ref, wout_ref, z_ref):
    s = agg_ref[0] + agg_ref[1]
    h2 = jnp.maximum(s * nd_ref[...] + b_ref[...], 0.0)
    z_ref[...] = (
        jnp.dot(h1_ref[...], wout_ref[:H], preferred_element_type=jnp.float32,
                precision=_PREC)
        + jnp.dot(h2, wout_ref[H:], preferred_element_type=jnp.float32,
                  precision=_PREC)
    )


def _tc4_body(agg_ref, b_ref, out_ref):
    out_ref[...] = agg_ref[0] + agg_ref[1] + b_ref[...]


def _row_blk(i):
    return (i, 0)


def _agg_blk(i):
    return (0, i, 0)


def _full_blk(i):
    return (0, 0)


_ROWS_SPEC = pl.BlockSpec((NPB, 128), _row_blk)
_COL_SPEC = pl.BlockSpec((NPB, 1), _row_blk)
_AGG_SPEC = pl.BlockSpec((2, NPB, 128), _agg_blk)
_B_SPEC = pl.BlockSpec((1, 128), _full_blk)


def _tc0(x_p, W1):
    return pl.pallas_call(
        _tc0_body,
        grid=(2,),
        in_specs=[_ROWS_SPEC, pl.BlockSpec((128, 128), _full_blk)],
        out_specs=_ROWS_SPEC,
        out_shape=jax.ShapeDtypeStruct((NP, 128), jnp.float32),
    )(x_p, W1)


def _tc1(degp, u1):
    return pl.pallas_call(
        _tc1_body,
        grid=(2,),
        in_specs=[
            pl.BlockSpec((NW, 2, NPB), lambda i: (0, 0, i)),
            _ROWS_SPEC,
        ],
        out_specs=[_ROWS_SPEC, _COL_SPEC, _COL_SPEC],
        out_shape=[
            jax.ShapeDtypeStruct((NP, 128), jnp.float32),
            jax.ShapeDtypeStruct((NP, 1), jnp.float32),
            jax.ShapeDtypeStruct((NP, 1), jnp.float32),
        ],
    )(degp, u1)


def _tc2(agg, nd, ns, b, W):
    return pl.pallas_call(
        _tc2_body,
        grid=(2,),
        in_specs=[_AGG_SPEC, _COL_SPEC, _COL_SPEC, _B_SPEC,
                  pl.BlockSpec((128, 128), _full_blk)],
        out_specs=[_ROWS_SPEC, _ROWS_SPEC],
        out_shape=[
            jax.ShapeDtypeStruct((NP, 128), jnp.float32),
            jax.ShapeDtypeStruct((NP, 128), jnp.float32),
        ],
    )(agg, nd, ns, b, W)


def _tc3(agg, nd, b, h1, Wout):
    return pl.pallas_call(
        _tc3_body,
        grid=(2,),
        in_specs=[_AGG_SPEC, _COL_SPEC, _B_SPEC, _ROWS_SPEC,
                  pl.BlockSpec((2 * H, 128), _full_blk)],
        out_specs=_ROWS_SPEC,
        out_shape=jax.ShapeDtypeStruct((NP, 128), jnp.float32),
    )(agg, nd, b, h1, Wout)


def _tc4(agg, b):
    return pl.pallas_call(
        _tc4_body,
        grid=(5,),
        in_specs=[pl.BlockSpec((2, N // 5, 128), lambda i: (0, i, 0)), _B_SPEC],
        out_specs=pl.BlockSpec((N // 5, 128), _row_blk),
        out_shape=jax.ShapeDtypeStruct((N, 128), jnp.float32),
    )(agg, b)


def kernel(x, edge_index, W1, b1, W2, b2, Wout, bout):
    src = edge_index[0]
    dst = edge_index[1]

    pad = (N + (jnp.arange(EP - E, dtype=jnp.int32) % (NP - N))).astype(jnp.int32)
    src_p = jnp.concatenate([src, pad]).reshape(ROWS, B)
    dst_p = jnp.concatenate([dst, pad]).reshape(ROWS, B)
    x_p = jnp.pad(x, ((0, NP - N), (0, 0)))

    deg_k, ep_k = _make_sc_kernels()

    u1 = _tc0(x_p, W1)
    degp = deg_k(src_p, dst_p).reshape(NW, 2, NP)
    q1, ns, nd = _tc1(degp, u1)
    agg1 = ep_k(q1, src_p, dst_p)
    h1, q2 = _tc2(agg1, nd, ns, b1.reshape(1, H), W2)
    agg2 = ep_k(q2, src_p, dst_p)
    z = _tc3(agg2, nd, b2.reshape(1, H), h1, Wout)
    agg3 = ep_k(z, src_p, dst_p)
    return _tc4(agg3, bout.reshape(1, OUT))


# speedup vs baseline: 15.4787x; 1.0012x over previous
import jax
import jax.numpy as jnp
from jax import lax
from jax.experimental import pallas as pl
from jax.experimental.pallas import tpu as pltpu
from jax.experimental.pallas import tpu_sc as plsc

N = 10000
D = 128
E = 320000
H = 128
OUT = 128

NW = 32
B = 128
BPT = 80
ROWS = NW * BPT
EP = ROWS * B

NSUB = 16
RPS = 640
NP = NSUB * RPS
WCH = RPS // B
NPB = NP // 2


def _zero_rows(ref, nrows):
    zeros16 = jnp.zeros((16,), jnp.float32)

    def body(r, _):
        for o in range(8):
            ref[r, pl.ds(o * 16, 16)] = zeros16
        return 0

    lax.fori_loop(0, nrows, body, 0)


def _deg_body(sidx_hbm, didx_hbm, out_hbm, sidx_v, didx_v, acc_v):
    cid = lax.axis_index("c")
    sid = lax.axis_index("s")
    wid = sid * 2 + cid

    def zb(i, _):
        for u in range(8):
            acc_v[pl.ds(i * 128 + u * 16, 16)] = jnp.zeros((16,), jnp.float32)
        return 0

    lax.fori_loop(0, (2 * NP) // 128, zb, 0)

    pltpu.sync_copy(sidx_hbm.at[pl.ds(wid * BPT, BPT)], sidx_v)
    pltpu.sync_copy(didx_hbm.at[pl.ds(wid * BPT, BPT)], didx_v)

    ones16 = jnp.ones((16,), jnp.float32)

    npv = jnp.full((16,), NP, jnp.int32)

    def body(j, _):
        for o in range(8):
            plsc.addupdate_scatter(acc_v, [sidx_v[j, pl.ds(o * 16, 16)]], ones16)
            plsc.addupdate_scatter(acc_v, [didx_v[j, pl.ds(o * 16, 16)] + npv],
                                   ones16)
        return 0

    lax.fori_loop(0, BPT, body, 0)
    pltpu.sync_copy(acc_v, out_hbm.at[pl.ds(wid * 2 * NP, 2 * NP)])


IG = 16
NCH = BPT // IG


def _ep_body(q_hbm, sidx_hbm, didx_hbm, out_hbm, sidx_v, di0, di1,
             rv0, rv1, acc_sh, gs0, gs1, isem, zsem):
    cid = lax.axis_index("c")
    sid = lax.axis_index("s")
    wid = sid * 2 + cid
    rows = (rv0, rv1)
    gsems = (gs0, gs1)
    dibufs = (di0, di1)

    pltpu.async_copy(sidx_hbm.at[pl.ds(wid * BPT, BPT)], sidx_v, gs0)
    pltpu.async_copy(didx_hbm.at[pl.ds(wid * BPT, IG)], di0, gs1)

    _zero_rows(rv0, B)
    for k in range(WCH):
        pltpu.async_copy(rv0, acc_sh.at[pl.ds(sid * RPS + k * B, B)], zsem)
    for k in range(WCH):
        pltpu.make_async_copy(rv0, acc_sh.at[pl.ds(0, B)], zsem).wait()
    plsc.subcore_barrier()
    pltpu.make_async_copy(sidx_hbm.at[pl.ds(0, BPT)], sidx_v, gs0).wait()
    pltpu.make_async_copy(didx_hbm.at[pl.ds(0, IG)], di0, gs1).wait()

    def gather(t, b):
        pltpu.async_copy(q_hbm.at[sidx_v.at[t]], rows[b], gsems[b])

    def gwait(b):
        pltpu.make_async_copy(q_hbm.at[sidx_v.at[0]], rows[b], gsems[b]).wait()

    def scatter(r, di, b):
        pltpu.sync_copy(rows[b], acc_sh.at[di.at[r]], add=True)

    def load_chunk(c):
        pltpu.async_copy(didx_hbm.at[pl.ds(wid * BPT + c * IG, IG)],
                         dibufs[c % 2], isem)

    def chunk_wait(c):
        pltpu.make_async_copy(didx_hbm.at[pl.ds(0, IG)], dibufs[c % 2],
                              isem).wait()

    gather(0, 0)
    gather(1, 1)
    load_chunk(1)

    for c in range(NCH):
        di = dibufs[c % 2]
        base = c * IG
        if c >= 1:
            chunk_wait(c)
            if c + 1 < NCH:
                load_chunk(c + 1)

        last = c == NCH - 1
        hi = IG // 2 - 1 if last else IG // 2

        def cpair(i, _, di=di, base=base):
            r = 2 * i
            t = base + r
            gwait(0); scatter(r, di, 0); gather(t + 2, 0)
            gwait(1); scatter(r + 1, di, 1); gather(t + 3, 1)
            return 0

        lax.fori_loop(0, hi, cpair, 0)
        if last:
            gwait(0); scatter(IG - 2, di, 0)
            gwait(1); scatter(IG - 1, di, 1)

    plsc.subcore_barrier()

    for k in range(WCH):
        b = k % 2
        sl = pl.ds(sid * RPS + k * B, B)
        if k >= 2:
            pltpu.make_async_copy(rows[b], out_hbm.at[cid, pl.ds(0, B)],
                                  gsems[b]).wait()
        pltpu.sync_copy(acc_sh.at[sl], rows[b])
        pltpu.async_copy(rows[b], out_hbm.at[cid, sl], gsems[b])
    pltpu.make_async_copy(rv0, out_hbm.at[cid, pl.ds(0, B)], gsems[(WCH - 2) % 2]).wait()
    pltpu.make_async_copy(rv1, out_hbm.at[cid, pl.ds(0, B)], gsems[(WCH - 1) % 2]).wait()


def _make_sc_kernels():
    mesh = plsc.VectorSubcoreMesh(core_axis_name="c", subcore_axis_name="s")
    params = pltpu.CompilerParams(needs_layout_passes=False)
    deg = pl.kernel(
        _deg_body,
        out_type=jax.ShapeDtypeStruct((NW * 2 * NP,), jnp.float32),
        mesh=mesh,
        compiler_params=params,
        scratch_types=[
            pltpu.VMEM((BPT, B), jnp.int32),
            pltpu.VMEM((BPT, B), jnp.int32),
            pltpu.VMEM((2 * NP,), jnp.float32),
        ],
    )
    ep = pl.kernel(
        _ep_body,
        out_type=jax.ShapeDtypeStruct((2, NP, 128), jnp.float32),
        mesh=mesh,
        compiler_params=params,
        scratch_types=(
            [pltpu.VMEM((BPT, B), jnp.int32)]
            + [pltpu.VMEM((IG, B), jnp.int32)] * 2
            + [pltpu.VMEM((B, 128), jnp.float32)] * 2
            + [pltpu.VMEM_SHARED((NP, 128), jnp.float32)]
            + [pltpu.SemaphoreType.DMA] * 4
        ),
    )
    return deg, ep


_PREC = None


def _tc0_body(x_ref, w1_ref, u1_ref):
    u1_ref[...] = jnp.dot(x_ref[...], w1_ref[...],
                          preferred_element_type=jnp.float32, precision=_PREC)


def _tc1_body(degp_ref, u1_ref, q1_ref, ns_ref, nd_ref):
    deg = jnp.sum(degp_ref[...], axis=0)
    ns = lax.rsqrt(jnp.maximum(deg[0], 1.0))[:, None]
    nd = lax.rsqrt(jnp.maximum(deg[1], 1.0))[:, None]
    ns_ref[...] = ns
    nd_ref[...] = nd
    q1_ref[...] = u1_ref[...] * ns


def _tc2_body(agg_ref, nd_ref, ns_ref, b_ref, w_ref, h_ref, q_ref):
    s = agg_ref[0] + agg_ref[1]
    h = jnp.maximum(s * nd_ref[...] + b_ref[...], 0.0)
    h_ref[...] = h
    q_ref[...] = jnp.dot(h * ns_ref[...], w_ref[...],
                         preferred_element_type=jnp.float32, precision=_PREC)


def _tc3_body(agg_ref, nd_ref, b_ref, h1_ref, wout_ref, z_ref):
    s = agg_ref[0] + agg_ref[1]
    h2 = jnp.maximum(s * nd_ref[...] + b_ref[...], 0.0)
    z_ref[...] = (
        jnp.dot(h1_ref[...], wout_ref[:H], preferred_element_type=jnp.float32,
                precision=_PREC)
        + jnp.dot(h2, wout_ref[H:], preferred_element_type=jnp.float32,
                  precision=_PREC)
    )


def _tc4_body(agg_ref, b_ref, out_ref):
    out_ref[...] = agg_ref[0] + agg_ref[1] + b_ref[...]


def _row_blk(i):
    return (i, 0)


def _agg_blk(i):
    return (0, i, 0)


def _full_blk(i):
    return (0, 0)


_ROWS_SPEC = pl.BlockSpec((NPB, 128), _row_blk)
_COL_SPEC = pl.BlockSpec((NPB, 1), _row_blk)
_AGG_SPEC = pl.BlockSpec((2, NPB, 128), _agg_blk)
_B_SPEC = pl.BlockSpec((1, 128), _full_blk)


def _tc0(x_p, W1):
    return pl.pallas_call(
        _tc0_body,
        grid=(2,),
        in_specs=[_ROWS_SPEC, pl.BlockSpec((128, 128), _full_blk)],
        out_specs=_ROWS_SPEC,
        out_shape=jax.ShapeDtypeStruct((NP, 128), jnp.float32),
    )(x_p, W1)


def _tc1(degp, u1):
    return pl.pallas_call(
        _tc1_body,
        grid=(2,),
        in_specs=[
            pl.BlockSpec((NW, 2, NPB), lambda i: (0, 0, i)),
            _ROWS_SPEC,
        ],
        out_specs=[_ROWS_SPEC, _COL_SPEC, _COL_SPEC],
        out_shape=[
            jax.ShapeDtypeStruct((NP, 128), jnp.float32),
            jax.ShapeDtypeStruct((NP, 1), jnp.float32),
            jax.ShapeDtypeStruct((NP, 1), jnp.float32),
        ],
    )(degp, u1)


def _tc2(agg, nd, ns, b, W):
    return pl.pallas_call(
        _tc2_body,
        grid=(2,),
        in_specs=[_AGG_SPEC, _COL_SPEC, _COL_SPEC, _B_SPEC,
                  pl.BlockSpec((128, 128), _full_blk)],
        out_specs=[_ROWS_SPEC, _ROWS_SPEC],
        out_shape=[
            jax.ShapeDtypeStruct((NP, 128), jnp.float32),
            jax.ShapeDtypeStruct((NP, 128), jnp.float32),
        ],
    )(agg, nd, ns, b, W)


def _tc3(agg, nd, b, h1, Wout):
    return pl.pallas_call(
        _tc3_body,
        grid=(2,),
        in_specs=[_AGG_SPEC, _COL_SPEC, _B_SPEC, _ROWS_SPEC,
                  pl.BlockSpec((2 * H, 128), _full_blk)],
        out_specs=_ROWS_SPEC,
        out_shape=jax.ShapeDtypeStruct((NP, 128), jnp.float32),
    )(agg, nd, b, h1, Wout)


def _tc4(agg, b):
    return pl.pallas_call(
        _tc4_body,
        grid=(5,),
        in_specs=[pl.BlockSpec((2, N // 5, 128), lambda i: (0, i, 0)), _B_SPEC],
        out_specs=pl.BlockSpec((N // 5, 128), _row_blk),
        out_shape=jax.ShapeDtypeStruct((N, 128), jnp.float32),
    )(agg, b)


def kernel(x, edge_index, W1, b1, W2, b2, Wout, bout):
    src = edge_index[0]
    dst = edge_index[1]

    pad = (N + (jnp.arange(EP - E, dtype=jnp.int32) % (NP - N))).astype(jnp.int32)
    src_p = jnp.concatenate([src, pad]).reshape(ROWS, B)
    dst_p = jnp.concatenate([dst, pad]).reshape(ROWS, B)
    x_p = jnp.pad(x, ((0, NP - N), (0, 0)))

    deg_k, ep_k = _make_sc_kernels()

    u1 = _tc0(x_p, W1)
    degp = deg_k(src_p, dst_p).reshape(NW, 2, NP)
    q1, ns, nd = _tc1(degp, u1)
    agg1 = ep_k(q1, src_p, dst_p)
    h1, q2 = _tc2(agg1, nd, ns, b1.reshape(1, H), W2)
    agg2 = ep_k(q2, src_p, dst_p)
    z = _tc3(agg2, nd, b2.reshape(1, H), h1, Wout)
    agg3 = ep_k(z, src_p, dst_p)
    return _tc4(agg3, bout.reshape(1, OUT))
